# baseline (speedup 1.0000x reference)
import dataclasses
import functools

import jax
import jax.numpy as jnp
from jax.experimental import pallas as pl
from jax.experimental.pallas import tpu as pltpu
from jax.experimental.pallas import tpu_sc as plsc

N_NODES = 10000
N_EDGES = 320000
N_GRAPHS = 8
K = 5
EPS = 1e-5
NP = 10240
EB = 12800
NEB = N_EDGES // EB
BIG = 100.0

_HIGH = jax.lax.Precision.HIGHEST


def _dotT(a, b):
    return jax.lax.dot_general(a, b, (((0,), (0,)), ((), ())),
                               precision=_HIGH,
                               preferred_element_type=jnp.float32)


def _dot(a, b):
    return jax.lax.dot_general(a, b, (((1,), (0,)), ((), ())),
                               precision=_HIGH,
                               preferred_element_type=jnp.float32)


def _dotTT(a, b):
    return jax.lax.dot_general(a, b, (((0,), (1,)), ((), ())),
                               precision=_HIGH,
                               preferred_element_type=jnp.float32)


def _dot_bf16(a, b):
    return jax.lax.dot_general(a.astype(jnp.bfloat16),
                               b.astype(jnp.bfloat16),
                               (((1,), (0,)), ((), ())),
                               preferred_element_type=jnp.float32)


def _dotT_bf16(a, b):
    return jax.lax.dot_general(a.astype(jnp.bfloat16),
                               b.astype(jnp.bfloat16),
                               (((0,), (0,)), ((), ())),
                               preferred_element_type=jnp.float32)


def _sc_gather(table, idx_flat, window=128):
    n = idx_flat.shape[0]
    idx2 = idx_flat.reshape(1, n)
    mesh = plsc.VectorSubcoreMesh(core_axis_name="core",
                                  subcore_axis_name="subcore")

    @pl.kernel(out_type=jax.ShapeDtypeStruct((n, table.shape[1]),
                                             table.dtype),
               mesh=mesh)
    def k(tab_hbm, i_hbm, o_hbm):
        def body(i_vmem, o_vmem):
            pltpu.sync_copy(tab_hbm.at[i_vmem.at[0]], o_vmem)

        pltpu.emit_pipeline(
            body,
            grid=(n // window,),
            in_specs=[pl.BlockSpec((1, window), index_map=lambda i: (0, i))],
            out_specs=[pl.BlockSpec((window, table.shape[1]),
                                    index_map=lambda i: (i, 0))],
            core_axis_name="subcore",
            dimension_semantics=(pltpu.PARALLEL,),
        )(i_hbm, o_hbm)

    return k(table, idx2)


EP = 327680
_SEG_ROWS = 128
_SEG_NCH = EP // (128 * _SEG_ROWS)


def _sc_segmax(h3t3, dst3, ac3d):
    mesh = plsc.VectorSubcoreMesh(core_axis_name="core",
                                  subcore_axis_name="subcore")
    cp = pltpu.CompilerParams()
    if "needs_layout_passes" in pltpu.CompilerParams.__dataclass_fields__:
        cp = dataclasses.replace(cp, needs_layout_passes=False)

    @pl.kernel(out_type=jax.ShapeDtypeStruct((64, 80, 128), jnp.float32),
               mesh=mesh,
               compiler_params=cp,
               scratch_types=[pltpu.VMEM((80, 128), jnp.float32),
                              pltpu.VMEM((80, 128), jnp.float32),
                              pltpu.VMEM((2, _SEG_ROWS, 128), jnp.float32),
                              pltpu.VMEM((_SEG_ROWS, 128), jnp.int32),
                              pltpu.VMEM((2, 1, 128), jnp.float32),
                              pltpu.SemaphoreType.DMA,
                              pltpu.SemaphoreType.DMA,
                              pltpu.SemaphoreType.DMA])
    def k(h_hbm, d_hbm, ac_hbm, o_hbm, acc0, acc1, hbuf, dbuf, abuf,
          sem0, sem1, sem2):
        core = jax.lax.axis_index("core")
        sub = jax.lax.axis_index("subcore")
        fp = 2 * (core * 16 + sub)
        pltpu.async_copy(ac_hbm.at[pl.ds(fp, 2)], abuf, sem2).wait()

        @pl.loop(0, 80)
        def _(r):
            ninf = jnp.full((16,), -jnp.inf, jnp.float32)

            @pl.loop(0, 128, step=16)
            def _(l):
                acc0[r, pl.ds(l, 16)] = ninf
                acc1[r, pl.ds(l, 16)] = ninf

        av0 = abuf[0, 0, pl.ds(0, 16)]
        av1 = abuf[1, 0, pl.ds(0, 16)]
        a0, c0 = av0[0], av0[1]
        a1, c1 = av1[0], av1[1]

        @pl.loop(0, _SEG_NCH)
        def _(ci):
            r0 = ci * _SEG_ROWS
            cp1 = pltpu.async_copy(
                h_hbm.at[pl.ds(fp, 2), pl.ds(r0, _SEG_ROWS)], hbuf, sem0)
            cp2 = pltpu.async_copy(
                d_hbm.at[pl.ds(r0, _SEG_ROWS)], dbuf, sem1)
            cp1.wait()
            cp2.wait()

            @pl.loop(0, _SEG_ROWS)
            def _(r):
                @pl.loop(0, 128, step=16)
                def _(l):
                    idx = dbuf[r, pl.ds(l, 16)]
                    ri = jax.lax.shift_right_logical(idx, 7)
                    li = jax.lax.bitwise_and(idx, 127)

                    def rmw(acc, vals):
                        cur = plsc.load_gather(acc, [ri, li])
                        plsc.store_scatter(acc, [ri, li],
                                           jnp.maximum(cur, vals))
                        cur2 = plsc.load_gather(acc, [ri, li])

                        @pl.when(jnp.any(vals > cur2))
                        def _():
                            def body(_, c):
                                cur3 = plsc.load_gather(acc, [ri, li])
                                pend = vals > cur3
                                plsc.store_scatter(acc, [ri, li],
                                                   jnp.maximum(cur3, vals),
                                                   mask=pend)
                                return c
                            jax.lax.fori_loop(0, 15, body, jnp.int32(0))

                    rmw(acc0, hbuf[0, r, pl.ds(l, 16)] * a0 + c0)
                    rmw(acc1, hbuf[1, r, pl.ds(l, 16)] * a1 + c1)

        pltpu.async_copy(acc0, o_hbm.at[fp], sem0).wait()
        pltpu.async_copy(acc1, o_hbm.at[fp + 1], sem1).wait()

    return k(h3t3, dst3, ac3d)


def _k2_body(pd, ps, w1p, b1r, h1o, s1s, s1q):
    i = pl.program_id(0)

    @pl.when(i == 0)
    def _():
        s1s[...] = jnp.zeros_like(s1s)
        s1q[...] = jnp.zeros_like(s1q)

    xi = pd[...][:, :3]
    xj = ps[...][:, :3]
    m = jnp.concatenate([xi, xj - xi, jnp.zeros((EB, 122), jnp.float32)],
                        axis=1)
    h = jnp.maximum(_dot_bf16(m, w1p[...]) + b1r[:1, :], 0.0)
    h1o[...] = h
    s1s[...] += jnp.broadcast_to(jnp.sum(h, axis=0, keepdims=True), (8, 64))
    s1q[...] += jnp.broadcast_to(jnp.sum(h * h, axis=0, keepdims=True),
                                 (8, 64))


def _k34_body(hin, ss, sq, w, gr, ber, br, hout, oss, osq):
    i = pl.program_id(0)

    @pl.when(i == 0)
    def _():
        oss[...] = jnp.zeros_like(oss)
        osq[...] = jnp.zeros_like(osq)

    ecnt = float(N_EDGES)
    mu = ss[:1, :] / ecnt
    var = sq[:1, :] / ecnt - mu * mu
    inv = jax.lax.rsqrt(var + EPS)
    n = (hin[...] - mu) * (inv * gr[:1, :]) + ber[:1, :]
    h = jnp.maximum(_dot_bf16(n, w[...]) + br[:1, :], 0.0)
    hout[...] = h
    oss[...] += jnp.broadcast_to(jnp.sum(h, axis=0, keepdims=True), (8, 64))
    osq[...] += jnp.broadcast_to(jnp.sum(h * h, axis=0, keepdims=True),
                                 (8, 64))


def _k4t_body(hin, i64, hto):
    hto[...] = _dotTT(i64[...], hin[...])


def _k4b_body(ss, sq, gr, ber, i64, ac):
    ecnt = float(N_EDGES)
    mu = ss[:1, :] / ecnt
    var = sq[:1, :] / ecnt - mu * mu
    a = gr[:1, :] * jax.lax.rsqrt(var + EPS)
    c = ber[:1, :] - mu * a
    cols = [_dotTT(i64[...], a), _dotTT(i64[...], c)]
    lane = jax.lax.broadcasted_iota(jnp.int32, (64, 128), 1)
    out = jnp.zeros((64, 128), jnp.float32)
    for li, colv in enumerate(cols):
        out = jnp.where(lane == li, jnp.broadcast_to(colv, (64, 128)), out)
    ac[...] = out


def _k5t_body(x1t, i64, x1n):
    x = x1t[...]
    x = jnp.where(jnp.isfinite(x), x, 0.0)
    x1n[...] = _dotT(x, i64[...])


def _knn_body(xr, xc, bhr, bhc, idxo, rv, ri):
    j = pl.program_id(1)

    @pl.when(j == 0)
    def _():
        rv[...] = jnp.full((256, 128), jnp.inf, jnp.float32)
        ri[...] = jnp.zeros((256, 128), jnp.int32)

    a = xr[...]
    a = jnp.where(jnp.isfinite(a), a, 0.0)
    b = xc[...]
    b = jnp.where(jnp.isfinite(b), b, 0.0)
    ones = jnp.ones((128, 1), jnp.float32)
    sqr = _dotT(a * a, ones)
    sqc = jnp.sum(b * b, axis=0, keepdims=True)
    g = _dotT_bf16(a, b)
    same = jax.lax.dot_general(bhr[...], bhc[...],
                               (((1,), (1,)), ((), ())),
                               preferred_element_type=jnp.float32)
    d = sqr + sqc - 2.0 * g
    d = jnp.where(same > 0.5, d, jnp.inf)

    cv = jnp.concatenate([rv[...], d], axis=1)
    base = jax.lax.broadcasted_iota(jnp.int32, (256, 256), 1) + j * 256
    ci = jnp.concatenate([ri[...], base], axis=1)
    lane = jax.lax.broadcasted_iota(jnp.int32, (256, 384), 1)
    lane128 = jax.lax.broadcasted_iota(jnp.int32, (256, 128), 1)

    nv = jnp.full((256, 128), jnp.inf, jnp.float32)
    ni = jnp.zeros((256, 128), jnp.int32)
    for k in range(K):
        mv = jnp.min(cv, axis=1, keepdims=True)
        is_min = cv == mv
        sel = jnp.min(jnp.where(is_min, lane, 10000), axis=1, keepdims=True)
        selmask = lane == sel
        chosen = jnp.sum(jnp.where(selmask, ci, 0), axis=1, keepdims=True)
        nv = jnp.where(lane128 == k, jnp.broadcast_to(mv, (256, 128)), nv)
        ni = jnp.where(lane128 == k, jnp.broadcast_to(chosen, (256, 128)), ni)
        cv = jnp.where(selmask, jnp.inf, cv)

    rv[...] = nv
    ri[...] = ni
    idxo[...] = ni


def _k6a_body(x1n, xj2, w4, b4r, r, s4s, s4q):
    i = pl.program_id(0)

    @pl.when(i == 0)
    def _():
        s4s[...] = jnp.zeros_like(s4s)
        s4q[...] = jnp.zeros_like(s4q)

    xi2 = jnp.broadcast_to(x1n[...][:, None, :], (512, K, 64)).reshape(
        512 * K, 64)
    m2 = jnp.concatenate([xi2, xj2[...][:, :64] - xi2], axis=1)
    h = jnp.maximum(_dot_bf16(m2, w4[...]) + b4r[:1, :], 0.0)
    r[...] = h
    row = jax.lax.broadcasted_iota(jnp.int32, (512 * K, 128), 0) + i * 512 * K
    hm = jnp.where(row < N_NODES * K, h, 0.0)
    s4s[...] += jnp.broadcast_to(jnp.sum(hm, axis=0, keepdims=True), (8, 128))
    s4q[...] += jnp.broadcast_to(jnp.sum(hm * hm, axis=0, keepdims=True),
                                 (8, 128))


def _k6b_body(r, s4s, s4q, g4r, be4r, x1n, w5a, w5b, b5r, b1h,
              gmax, gmin, s5s, s5q):
    i = pl.program_id(0)

    @pl.when(i == 0)
    def _():
        gmax[...] = jnp.full((8, 1024), -jnp.inf, jnp.float32)
        gmin[...] = jnp.full((8, 1024), jnp.inf, jnp.float32)
        s5s[...] = jnp.zeros_like(s5s)
        s5q[...] = jnp.zeros_like(s5q)

    nk = float(N_NODES * K)
    mu = s4s[:1, :] / nk
    var = s4q[:1, :] / nk - mu * mu
    a4 = g4r[:1, :] * jax.lax.rsqrt(var + EPS)
    c4 = be4r[:1, :] - mu * a4
    n4 = a4 * r[...] + c4
    x2 = jnp.max(n4.reshape(512, K, 128), axis=1)

    h = (_dot_bf16(x1n[...], w5a[...]) + _dot_bf16(x2, w5b[...])
         + b5r[:1, :])
    h = jnp.maximum(h, 0.0)

    row = jax.lax.broadcasted_iota(jnp.int32, (512, 1024), 0) + i * 512
    hm = jnp.where(row < N_NODES, h, 0.0)
    s5s[...] += jnp.broadcast_to(jnp.sum(hm, axis=0, keepdims=True),
                                 (8, 1024))
    s5q[...] += jnp.broadcast_to(jnp.sum(hm * hm, axis=0, keepdims=True),
                                 (8, 1024))

    srow = jax.lax.broadcasted_iota(jnp.int32, (8, 1024), 0)
    bh = b1h[...]
    for gr in range(N_GRAPHS):
        m = bh[:, gr:gr + 1] > 0.5
        mx = jnp.max(jnp.where(m, h, -jnp.inf), axis=0, keepdims=True)
        mn = jnp.min(jnp.where(m, h, jnp.inf), axis=0, keepdims=True)
        gmax[...] = jnp.where(srow == gr,
                              jnp.maximum(gmax[...],
                                          jnp.broadcast_to(mx, (8, 1024))),
                              gmax[...])
        gmin[...] = jnp.where(srow == gr,
                              jnp.minimum(gmin[...],
                                          jnp.broadcast_to(mn, (8, 1024))),
                              gmin[...])


def _k7_body(gmax, gmin, s5s, s5q, g5r, be5r, w6, b6r, g6r, be6r,
             w7, b7r, g7r, be7r, w8, b8r, out):
    nn = float(N_NODES)
    mu5 = s5s[:1, :] / nn
    var5 = s5q[:1, :] / nn - mu5 * mu5
    a5 = g5r[:1, :] * jax.lax.rsqrt(var5 + EPS)
    c5 = be5r[:1, :] - mu5 * a5
    gx = gmax[...]
    gn = gmin[...]
    hsel = jnp.where(a5 > 0, a5 * gx + c5,
                     jnp.where(a5 < 0, a5 * gn + c5,
                               jnp.broadcast_to(c5, (8, 1024))))
    h = jnp.where(gx == -jnp.inf, 0.0, hsel)

    def block(x, w, br, gr, ber):
        z = jnp.maximum(_dot_bf16(x, w) + br[:1, :], 0.0)
        mu = jnp.mean(z, axis=0, keepdims=True)
        var = jnp.mean(z * z, axis=0, keepdims=True) - mu * mu
        return (z - mu) * jax.lax.rsqrt(var + EPS) * gr[:1, :] + ber[:1, :]

    h = block(h, w6[...], b6r, g6r, be6r)
    h = block(h, w7[...], b7r, g7r, be7r)
    out[...] = _dot_bf16(h, w8[...]) + b8r[:1, :]


def _var_pass(hin, ss):
    return pl.pallas_call(
        _var_body,
        grid=(NEB,),
        in_specs=[
            pl.BlockSpec((EB, 64), lambda i: (i, 0)),
            pl.BlockSpec((8, 64), lambda i: (0, 0)),
        ],
        out_specs=pl.BlockSpec((8, 64), lambda i: (0, 0)),
        out_shape=jax.ShapeDtypeStruct((8, 64), jnp.float32),
    )(hin, ss)


def _edge_layer(hin, ss, sq, w, g_prev, be_prev, b):
    return pl.pallas_call(
        _k34_body,
        grid=(NEB,),
        in_specs=[
            pl.BlockSpec((EB, 64), lambda i: (i, 0)),
            pl.BlockSpec((8, 64), lambda i: (0, 0)),
            pl.BlockSpec((8, 64), lambda i: (0, 0)),
            pl.BlockSpec((64, 64), lambda i: (0, 0)),
            pl.BlockSpec((8, 64), lambda i: (0, 0)),
            pl.BlockSpec((8, 64), lambda i: (0, 0)),
            pl.BlockSpec((8, 64), lambda i: (0, 0)),
        ],
        out_specs=[
            pl.BlockSpec((EB, 64), lambda i: (i, 0)),
            pl.BlockSpec((8, 64), lambda i: (0, 0)),
            pl.BlockSpec((8, 64), lambda i: (0, 0)),
        ],
        out_shape=[
            jax.ShapeDtypeStruct((N_EDGES, 64), jnp.float32),
            jax.ShapeDtypeStruct((8, 64), jnp.float32),
            jax.ShapeDtypeStruct((8, 64), jnp.float32),
        ],
    )(hin, ss, sq, w, _bcast_row(g_prev), _bcast_row(be_prev),
      _bcast_row(b))


def _bcast_col(v, lanes=128):
    return jnp.broadcast_to(v[:, None], (v.shape[0], lanes))


def _bcast_row(v, rows=8):
    return jnp.broadcast_to(v[None, :], (rows, v.shape[0]))


@jax.jit
def kernel(pos, params, batch, edge_index):
    p = params
    f32 = jnp.float32
    src = edge_index[0]
    dst = edge_index[1]

    pos128 = jnp.pad(pos, ((0, 0), (0, 125)))
    idx_all = jnp.concatenate([dst, src]).reshape(640000)
    w1p = jnp.pad(p["W1"], ((0, 122), (0, 0)))
    i64 = jnp.eye(64, dtype=f32)

    gath = _sc_gather(pos128, idx_all)

    h1, s1s, s1q = pl.pallas_call(
        _k2_body,
        grid=(NEB,),
        in_specs=[
            pl.BlockSpec((EB, 128), lambda i: (i, 0)),
            pl.BlockSpec((EB, 128), lambda i: (i + NEB, 0)),
            pl.BlockSpec((128, 64), lambda i: (0, 0)),
            pl.BlockSpec((8, 64), lambda i: (0, 0)),
        ],
        out_specs=[
            pl.BlockSpec((EB, 64), lambda i: (i, 0)),
            pl.BlockSpec((8, 64), lambda i: (0, 0)),
            pl.BlockSpec((8, 64), lambda i: (0, 0)),
        ],
        out_shape=[
            jax.ShapeDtypeStruct((N_EDGES, 64), f32),
            jax.ShapeDtypeStruct((8, 64), f32),
            jax.ShapeDtypeStruct((8, 64), f32),
        ],
    )(gath, gath, w1p, _bcast_row(p["b1"]))

    h2, s2s, s2q = _edge_layer(h1, s1s, s1q, p["W2"], p["g1"], p["be1"],
                               p["b2"])
    h3, s3s, s3q = _edge_layer(h2, s2s, s2q, p["W3"], p["g2"], p["be2"],
                               p["b3"])

    h3t = pl.pallas_call(
        _k4t_body,
        grid=(NEB,),
        in_specs=[
            pl.BlockSpec((EB, 64), lambda i: (i, 0)),
            pl.BlockSpec((64, 64), lambda i: (0, 0)),
        ],
        out_specs=pl.BlockSpec((64, EB), lambda i: (0, i)),
        out_shape=jax.ShapeDtypeStruct((64, N_EDGES), f32),
    )(h3, i64)

    ac3 = pl.pallas_call(
        _k4b_body,
        grid=(1,),
        in_specs=[pl.BlockSpec((8, 64), lambda i: (0, 0))] * 4
        + [pl.BlockSpec((64, 64), lambda i: (0, 0))],
        out_specs=pl.BlockSpec((64, 128), lambda i: (0, 0)),
        out_shape=jax.ShapeDtypeStruct((64, 128), f32),
    )(s3s, s3q, _bcast_row(p["g3"]), _bcast_row(p["be3"]), i64)

    h3t3 = jnp.pad(h3t, ((0, 0), (0, EP - N_EDGES))).reshape(64, 2560, 128)
    dst3 = jnp.pad(dst, (0, EP - N_EDGES),
                   constant_values=NP - 1).reshape(2560, 128)
    x1t3 = _sc_segmax(h3t3, dst3, ac3.reshape(64, 1, 128))
    x1t_pad = x1t3.reshape(64, NP)

    x1n = pl.pallas_call(
        _k5t_body,
        grid=(NP // 256,),
        in_specs=[
            pl.BlockSpec((64, 256), lambda i: (0, i)),
            pl.BlockSpec((64, 64), lambda i: (0, 0)),
        ],
        out_specs=pl.BlockSpec((256, 64), lambda i: (i, 0)),
        out_shape=jax.ShapeDtypeStruct((NP, 64), f32),
    )(x1t_pad, i64)

    b1h = jnp.pad(jax.nn.one_hot(batch, N_GRAPHS, dtype=f32),
                  ((0, NP - N_NODES), (0, 0)))
    x1a = jnp.pad(x1t_pad, ((0, 64), (0, 0)))

    idx_knn = pl.pallas_call(
        _knn_body,
        grid=(NP // 256, NP // 256),
        in_specs=[
            pl.BlockSpec((128, 256), lambda i, j: (0, i)),
            pl.BlockSpec((128, 256), lambda i, j: (0, j)),
            pl.BlockSpec((256, 8), lambda i, j: (i, 0)),
            pl.BlockSpec((256, 8), lambda i, j: (j, 0)),
        ],
        out_specs=pl.BlockSpec((256, 128), lambda i, j: (i, 0)),
        out_shape=jax.ShapeDtypeStruct((NP, 128), jnp.int32),
        scratch_shapes=[pltpu.VMEM((256, 128), f32),
                        pltpu.VMEM((256, 128), jnp.int32)],
    )(x1a, x1a, b1h, b1h)

    idx5 = jnp.clip(idx_knn[:N_NODES, :K], 0, N_NODES - 1).reshape(-1)
    idx5 = jnp.pad(idx5, (0, NP * K - N_NODES * K))

    x1g = jnp.pad(x1n[:N_NODES], ((0, 0), (0, 64)))
    xj2 = _sc_gather(x1g, idx5)

    rmat, s4s, s4q = pl.pallas_call(
        _k6a_body,
        grid=(NP // 512,),
        in_specs=[
            pl.BlockSpec((512, 64), lambda i: (i, 0)),
            pl.BlockSpec((512 * K, 128), lambda i: (i, 0)),
            pl.BlockSpec((128, 128), lambda i: (0, 0)),
            pl.BlockSpec((8, 128), lambda i: (0, 0)),
        ],
        out_specs=[
            pl.BlockSpec((512 * K, 128), lambda i: (i, 0)),
            pl.BlockSpec((8, 128), lambda i: (0, 0)),
            pl.BlockSpec((8, 128), lambda i: (0, 0)),
        ],
        out_shape=[
            jax.ShapeDtypeStruct((NP * K, 128), f32),
            jax.ShapeDtypeStruct((8, 128), f32),
            jax.ShapeDtypeStruct((8, 128), f32),
        ],
    )(x1n, xj2, p["W4"], _bcast_row(p["b4"]))

    gmax, gmin, s5s, s5q = pl.pallas_call(
        _k6b_body,
        grid=(NP // 512,),
        in_specs=[
            pl.BlockSpec((512 * K, 128), lambda i: (i, 0)),
            pl.BlockSpec((8, 128), lambda i: (0, 0)),
            pl.BlockSpec((8, 128), lambda i: (0, 0)),
            pl.BlockSpec((8, 128), lambda i: (0, 0)),
            pl.BlockSpec((8, 128), lambda i: (0, 0)),
            pl.BlockSpec((512, 64), lambda i: (i, 0)),
            pl.BlockSpec((64, 1024), lambda i: (0, 0)),
            pl.BlockSpec((128, 1024), lambda i: (0, 0)),
            pl.BlockSpec((8, 1024), lambda i: (0, 0)),
            pl.BlockSpec((512, 8), lambda i: (i, 0)),
        ],
        out_specs=[
            pl.BlockSpec((8, 1024), lambda i: (0, 0)),
            pl.BlockSpec((8, 1024), lambda i: (0, 0)),
            pl.BlockSpec((8, 1024), lambda i: (0, 0)),
            pl.BlockSpec((8, 1024), lambda i: (0, 0)),
        ],
        out_shape=[
            jax.ShapeDtypeStruct((8, 1024), f32),
            jax.ShapeDtypeStruct((8, 1024), f32),
            jax.ShapeDtypeStruct((8, 1024), f32),
            jax.ShapeDtypeStruct((8, 1024), f32),
        ],
    )(rmat, s4s, s4q, _bcast_row(p["g4"]), _bcast_row(p["be4"]),
      x1n, p["W5"][:64], p["W5"][64:], _bcast_row(p["b5"]), b1h)

    w8p = jnp.pad(p["W8"], ((0, 0), (0, 128 - 40)))
    b8p = jnp.pad(p["b8"], (0, 128 - 40))
    out = pl.pallas_call(
        _k7_body,
        grid=(1,),
        in_specs=[
            pl.BlockSpec((8, 1024), lambda i: (0, 0)),
            pl.BlockSpec((8, 1024), lambda i: (0, 0)),
            pl.BlockSpec((8, 1024), lambda i: (0, 0)),
            pl.BlockSpec((8, 1024), lambda i: (0, 0)),
            pl.BlockSpec((8, 1024), lambda i: (0, 0)),
            pl.BlockSpec((8, 1024), lambda i: (0, 0)),
            pl.BlockSpec((1024, 512), lambda i: (0, 0)),
            pl.BlockSpec((8, 512), lambda i: (0, 0)),
            pl.BlockSpec((8, 512), lambda i: (0, 0)),
            pl.BlockSpec((8, 512), lambda i: (0, 0)),
            pl.BlockSpec((512, 256), lambda i: (0, 0)),
            pl.BlockSpec((8, 256), lambda i: (0, 0)),
            pl.BlockSpec((8, 256), lambda i: (0, 0)),
            pl.BlockSpec((8, 256), lambda i: (0, 0)),
            pl.BlockSpec((256, 128), lambda i: (0, 0)),
            pl.BlockSpec((8, 128), lambda i: (0, 0)),
        ],
        out_specs=pl.BlockSpec((8, 128), lambda i: (0, 0)),
        out_shape=jax.ShapeDtypeStruct((8, 128), f32),
    )(gmax, gmin, s5s, s5q, _bcast_row(p["g5"]), _bcast_row(p["be5"]),
      p["W6"], _bcast_row(p["b6"]), _bcast_row(p["g6"]), _bcast_row(p["be6"]),
      p["W7"], _bcast_row(p["b7"]), _bcast_row(p["g7"]), _bcast_row(p["be7"]),
      w8p, b8p[None, :].repeat(8, axis=0))

    return out[:, :40]
